# baseline (speedup 1.0000x reference)
import functools

import jax
import jax.numpy as jnp
from jax import lax
from jax.experimental import pallas as pl
from jax.experimental.pallas import tpu as pltpu
from jax.experimental.pallas import tpu_sc as plsc

NH = 3
INV_SQRT2 = 0.7071067811865475

E_TOTAL = 160000
CHUNK_SIZES = (64000, 48000, 48000)
NCHUNKS = len(CHUNK_SIZES)
CHUNK_OFFS = (0, 64000, 112000)
LOADW = 128


def _silu_half(y):
    return y + y * jnp.tanh(y)


def _res_mlp(t16, wres_ref, wout_ref):
    t = t16.astype(jnp.float32)
    for i in range(NH):
        y = _silu_half(jnp.dot(t16, wres_ref[i, 0],
                               preferred_element_type=jnp.float32
                               ).astype(jnp.bfloat16))
        y = _silu_half(jnp.dot(y, wres_ref[i, 1],
                               preferred_element_type=jnp.float32
                               ).astype(jnp.bfloat16))
        t = (t + y.astype(jnp.float32)) * INV_SQRT2
        t16 = t.astype(jnp.bfloat16)
    return jnp.dot(t16, wout_ref[...], preferred_element_type=jnp.float32)


def _fused_body(m_ref, rbf_ref, wr_ref, w1_ref, wres_ref, wout_ref,
                x_ref, o_ref):
    prod = jnp.dot(rbf_ref[...], wr_ref[...],
                   preferred_element_type=jnp.float32)
    xb = m_ref[...] * prod
    x_ref[...] = xb
    t16 = _silu_half(jnp.dot(xb.astype(jnp.bfloat16), w1_ref[...],
                             preferred_element_type=jnp.float32
                             ).astype(jnp.bfloat16))
    o_ref[...] = _res_mlp(t16, wres_ref, wout_ref)


def _fused_stage(m, rbf, w_rbf, w1, wres, wout, chunk, block):
    dE = m.shape[1]
    dR = rbf.shape[1]
    T = wout.shape[1]
    e_chunk = CHUNK_SIZES[chunk]
    grid = (e_chunk // block,)
    off = CHUNK_OFFS[chunk] // block
    return pl.pallas_call(
        _fused_body,
        grid=grid,
        in_specs=[
            pl.BlockSpec((block, dE), lambda i: (i + off, 0)),
            pl.BlockSpec((block, dR), lambda i: (i + off, 0)),
            pl.BlockSpec((dR, dE), lambda i: (0, 0)),
            pl.BlockSpec((dE, dE), lambda i: (0, 0)),
            pl.BlockSpec((NH, 2, dE, dE), lambda i: (0, 0, 0, 0)),
            pl.BlockSpec((dE, T), lambda i: (0, 0)),
        ],
        out_specs=[
            pl.BlockSpec((block, dE), lambda i: (i, 0)),
            pl.BlockSpec((block, T), lambda i: (i, 0)),
        ],
        out_shape=[
            jax.ShapeDtypeStruct((e_chunk, dE), jnp.float32),
            jax.ShapeDtypeStruct((e_chunk, T), jnp.float32),
        ],
    )(m, rbf, w_rbf, w1, wres, wout)


def _energy_body(*refs):
    xs_refs = refs[:NCHUNKS]
    w1_ref, wres_ref, wout_ref, o_ref = refs[NCHUNKS:]
    xs = xs_refs[0][...]
    for r in xs_refs[1:]:
        xs = xs + r[...]
    t16 = _silu_half(jnp.dot(xs.astype(jnp.bfloat16), w1_ref[...],
                             preferred_element_type=jnp.float32
                             ).astype(jnp.bfloat16))
    o_ref[...] = _res_mlp(t16, wres_ref, wout_ref)


def _energy_stage(xs_parts, w1, wres, wout, block):
    R, D = xs_parts[0].shape
    T = wout.shape[1]
    grid = (R // block,)
    return pl.pallas_call(
        _energy_body,
        grid=grid,
        in_specs=(
            [pl.BlockSpec((block, D), lambda i: (i, 0))
             for _ in range(NCHUNKS)] +
            [pl.BlockSpec((D, D), lambda i: (0, 0)),
             pl.BlockSpec((NH, 2, D, D), lambda i: (0, 0, 0, 0)),
             pl.BlockSpec((D, T), lambda i: (0, 0))]
        ),
        out_specs=pl.BlockSpec((block, T), lambda i: (i, 0)),
        out_shape=jax.ShapeDtypeStruct((R, T), jnp.float32),
    )(*xs_parts, w1, wres, wout)


def _segsum(x, ids_main, ids_tail, nAtoms):
    E_c, dE = x.shape
    half = dE // 2
    ept = E_c // 16
    nmain = ept // LOADW
    tailw = ept - nmain * LOADW
    rpt = 624
    tail0 = 15 * rpt
    tail_n = nAtoms - tail0

    mesh = plsc.VectorSubcoreMesh(core_axis_name="c", subcore_axis_name="s")

    @functools.partial(
        pl.kernel,
        out_type=jax.ShapeDtypeStruct((nAtoms, dE), jnp.float32),
        mesh=mesh,
        scratch_types=[
            pltpu.VMEM((nmain, LOADW), jnp.int32),
            pltpu.VMEM((tailw,), jnp.int32),
            pltpu.VMEM((LOADW, half), jnp.float32),
            pltpu.VMEM((LOADW, half), jnp.float32),
            pltpu.VMEM_SHARED((nAtoms, half), jnp.float32),
            pltpu.SemaphoreType.DMA,
            pltpu.SemaphoreType.DMA,
        ],
    )
    def k(x_hbm, idm_hbm, idt_hbm, out_hbm, ids_v, idt_v, xb0, xb1, acc,
          sem0, sem1):
        c = lax.axis_index("c")
        s = lax.axis_index("s")
        xbufs = (xb0, xb1)
        sems = (sem0, sem1)

        @pl.loop(0, LOADW)
        def _(r):
            @pl.loop(0, half, step=16)
            def _(cc):
                xb0[r, pl.ds(cc, 16)] = jnp.zeros((16,), jnp.float32)

        row0 = s * rpt
        for z in range(4):
            pltpu.sync_copy(xb0, acc.at[pl.ds(row0 + z * LOADW, LOADW)])
        pltpu.sync_copy(xb0.at[pl.ds(0, rpt - 4 * LOADW)],
                        acc.at[pl.ds(row0 + 4 * LOADW, rpt - 4 * LOADW)])

        @pl.when(s == 15)
        def _():
            pltpu.sync_copy(xb0.at[pl.ds(0, tail_n - rpt)],
                            acc.at[pl.ds(tail0 + rpt, tail_n - rpt)])

        pltpu.sync_copy(idm_hbm.at[s], ids_v)
        pltpu.sync_copy(idt_hbm.at[s], idt_v)
        plsc.subcore_barrier()

        e_base = s * ept
        col = c * half

        def start_load(w):
            return pltpu.async_copy(
                x_hbm.at[pl.ds(e_base + w * LOADW, LOADW),
                         pl.ds(col, half)],
                xbufs[w % 2], sems[w % 2])

        handles = {0: start_load(0)}
        for w in range(nmain):
            if w + 1 < nmain:
                handles[w + 1] = start_load(w + 1)
            handles.pop(w).wait()
            pltpu.sync_copy(xbufs[w % 2], acc.at[ids_v.at[w]], add=True)

        pltpu.sync_copy(
            x_hbm.at[pl.ds(e_base + nmain * LOADW, tailw), pl.ds(col, half)],
            xb0.at[pl.ds(0, tailw)])
        pltpu.sync_copy(xb0.at[pl.ds(0, tailw)], acc.at[idt_v], add=True)

        plsc.subcore_barrier()
        pltpu.sync_copy(
            acc.at[pl.ds(row0, rpt)],
            out_hbm.at[pl.ds(row0, rpt), pl.ds(c * half, half)])

        @pl.when(s == 15)
        def _():
            pltpu.sync_copy(
                acc.at[pl.ds(tail0 + rpt, tail_n - rpt)],
                out_hbm.at[pl.ds(tail0 + rpt, tail_n - rpt),
                           pl.ds(c * half, half)])

    return k(x, ids_main, ids_tail)


def kernel(h, m, rbf, id_j, W_rbf, W1_E, Wres_E, W_out_E,
           W1_F, Wres_F, W_out_F, scale_sum, scale_rbf):
    nAtoms = h.shape[0]

    ids32 = id_j.astype(jnp.int32)
    wr16 = W_rbf.astype(jnp.bfloat16)
    rbf16 = rbf.astype(jnp.bfloat16)
    w1_f = (W1_F * (0.5 * scale_rbf)).astype(jnp.bfloat16)
    wres_f = (Wres_F * 0.5).astype(jnp.bfloat16)
    wout_f = W_out_F.astype(jnp.bfloat16)
    w1_e = (W1_E * (0.5 * scale_sum)).astype(jnp.bfloat16)
    wres_e = (Wres_E * 0.5).astype(jnp.bfloat16)
    wout_e = W_out_E.astype(jnp.bfloat16)

    xs_parts = []
    xf_parts = []
    for chunk in range(NCHUNKS):
        x_c, xf_c = _fused_stage(m, rbf16, wr16, w1_f, wres_f, wout_f,
                                 chunk, block=4000)
        e_c = CHUNK_SIZES[chunk]
        ept = e_c // 16
        nmain = ept // LOADW
        ids_c = lax.dynamic_slice_in_dim(ids32, CHUNK_OFFS[chunk], e_c
                                         ).reshape(16, ept)
        ids_main = ids_c[:, :nmain * LOADW].reshape(16, nmain, LOADW)
        ids_tail = ids_c[:, nmain * LOADW:]
        xs_parts.append(_segsum(x_c, ids_main, ids_tail, nAtoms))
        xf_parts.append(xf_c)

    x_E = _energy_stage(xs_parts, w1_e, wres_e, wout_e, block=2000)
    x_F = jnp.concatenate(xf_parts, axis=0)
    return (x_E, x_F)


# speedup vs baseline: 1.0078x; 1.0078x over previous
import functools

import jax
import jax.numpy as jnp
from jax import lax
from jax.experimental import pallas as pl
from jax.experimental.pallas import tpu as pltpu
from jax.experimental.pallas import tpu_sc as plsc

NH = 3
INV_SQRT2 = 0.7071067811865475

NCHUNKS = 2
E_TOTAL = 160000
E_CHUNK = E_TOTAL // NCHUNKS
EDGES_PER_TILE = E_CHUNK // 16
LOADW = 128
NMAIN = EDGES_PER_TILE // LOADW
TAILW = EDGES_PER_TILE - NMAIN * LOADW


def _silu_half(y):
    return y + y * jnp.tanh(y)


def _res_mlp(t16, wres_ref, wout_ref):
    t = t16.astype(jnp.float32)
    for i in range(NH):
        y = _silu_half(jnp.dot(t16, wres_ref[i, 0],
                               preferred_element_type=jnp.float32
                               ).astype(jnp.bfloat16))
        y = _silu_half(jnp.dot(y, wres_ref[i, 1],
                               preferred_element_type=jnp.float32
                               ).astype(jnp.bfloat16))
        t = (t + y.astype(jnp.float32)) * INV_SQRT2
        t16 = t.astype(jnp.bfloat16)
    return jnp.dot(t16, wout_ref[...], preferred_element_type=jnp.float32)


def _fused_body(m_ref, rbf_ref, wr_ref, w1_ref, wres_ref, wout_ref,
                x_ref, o_ref):
    prod = jnp.dot(rbf_ref[...], wr_ref[...],
                   preferred_element_type=jnp.float32)
    xb = m_ref[...] * prod
    x_ref[...] = xb
    t16 = _silu_half(jnp.dot(xb.astype(jnp.bfloat16), w1_ref[...],
                             preferred_element_type=jnp.float32
                             ).astype(jnp.bfloat16))
    o_ref[...] = _res_mlp(t16, wres_ref, wout_ref)


def _fused_stage(m, rbf, w_rbf, w1, wres, wout, chunk, block):
    dE = m.shape[1]
    dR = rbf.shape[1]
    T = wout.shape[1]
    grid = (E_CHUNK // block,)
    off = chunk * (E_CHUNK // block)
    return pl.pallas_call(
        _fused_body,
        grid=grid,
        in_specs=[
            pl.BlockSpec((block, dE), lambda i: (i + off, 0)),
            pl.BlockSpec((block, dR), lambda i: (i + off, 0)),
            pl.BlockSpec((dR, dE), lambda i: (0, 0)),
            pl.BlockSpec((dE, dE), lambda i: (0, 0)),
            pl.BlockSpec((NH, 2, dE, dE), lambda i: (0, 0, 0, 0)),
            pl.BlockSpec((dE, T), lambda i: (0, 0)),
        ],
        out_specs=[
            pl.BlockSpec((block, dE), lambda i: (i, 0)),
            pl.BlockSpec((block, T), lambda i: (i, 0)),
        ],
        out_shape=[
            jax.ShapeDtypeStruct((E_CHUNK, dE), jnp.float32),
            jax.ShapeDtypeStruct((E_CHUNK, T), jnp.float32),
        ],
    )(m, rbf, w_rbf, w1, wres, wout)


def _energy_body(*refs):
    xs_refs = refs[:NCHUNKS]
    w1_ref, wres_ref, wout_ref, o_ref = refs[NCHUNKS:]
    xs = xs_refs[0][...]
    for r in xs_refs[1:]:
        xs = xs + r[...]
    t16 = _silu_half(jnp.dot(xs.astype(jnp.bfloat16), w1_ref[...],
                             preferred_element_type=jnp.float32
                             ).astype(jnp.bfloat16))
    o_ref[...] = _res_mlp(t16, wres_ref, wout_ref)


def _energy_stage(xs_parts, w1, wres, wout, block):
    R, D = xs_parts[0].shape
    T = wout.shape[1]
    grid = (R // block,)
    return pl.pallas_call(
        _energy_body,
        grid=grid,
        in_specs=(
            [pl.BlockSpec((block, D), lambda i: (i, 0))
             for _ in range(NCHUNKS)] +
            [pl.BlockSpec((D, D), lambda i: (0, 0)),
             pl.BlockSpec((NH, 2, D, D), lambda i: (0, 0, 0, 0)),
             pl.BlockSpec((D, T), lambda i: (0, 0))]
        ),
        out_specs=pl.BlockSpec((block, T), lambda i: (i, 0)),
        out_shape=jax.ShapeDtypeStruct((R, T), jnp.float32),
    )(*xs_parts, w1, wres, wout)


def _segsum(x, ids_main, ids_tail, nAtoms):
    _, dE = x.shape
    half = dE // 2
    rpt = 624
    tail0 = 15 * rpt
    tail_n = nAtoms - tail0

    mesh = plsc.VectorSubcoreMesh(core_axis_name="c", subcore_axis_name="s")

    @functools.partial(
        pl.kernel,
        out_type=jax.ShapeDtypeStruct((nAtoms, dE), jnp.float32),
        mesh=mesh,
        scratch_types=[
            pltpu.VMEM((NMAIN, LOADW), jnp.int32),
            pltpu.VMEM((TAILW,), jnp.int32),
            pltpu.VMEM((LOADW, half), jnp.float32),
            pltpu.VMEM((LOADW, half), jnp.float32),
            pltpu.VMEM_SHARED((nAtoms, half), jnp.float32),
            pltpu.SemaphoreType.DMA,
            pltpu.SemaphoreType.DMA,
        ],
    )
    def k(x_hbm, idm_hbm, idt_hbm, out_hbm, ids_v, idt_v, xb0, xb1, acc,
          sem0, sem1):
        c = lax.axis_index("c")
        s = lax.axis_index("s")
        xbufs = (xb0, xb1)
        sems = (sem0, sem1)

        @pl.loop(0, LOADW)
        def _(r):
            @pl.loop(0, half, step=16)
            def _(cc):
                xb0[r, pl.ds(cc, 16)] = jnp.zeros((16,), jnp.float32)

        row0 = s * rpt
        for z in range(4):
            pltpu.sync_copy(xb0, acc.at[pl.ds(row0 + z * LOADW, LOADW)])
        pltpu.sync_copy(xb0.at[pl.ds(0, rpt - 4 * LOADW)],
                        acc.at[pl.ds(row0 + 4 * LOADW, rpt - 4 * LOADW)])

        @pl.when(s == 15)
        def _():
            pltpu.sync_copy(xb0.at[pl.ds(0, tail_n - rpt)],
                            acc.at[pl.ds(tail0 + rpt, tail_n - rpt)])

        pltpu.sync_copy(idm_hbm.at[s], ids_v)
        pltpu.sync_copy(idt_hbm.at[s], idt_v)
        plsc.subcore_barrier()

        e_base = s * EDGES_PER_TILE
        col = c * half

        def start_load(w):
            return pltpu.async_copy(
                x_hbm.at[pl.ds(e_base + w * LOADW, LOADW),
                         pl.ds(col, half)],
                xbufs[w % 2], sems[w % 2])

        handles = {0: start_load(0)}
        for w in range(NMAIN):
            if w + 1 < NMAIN:
                handles[w + 1] = start_load(w + 1)
            handles.pop(w).wait()
            pltpu.sync_copy(xbufs[w % 2], acc.at[ids_v.at[w]], add=True)

        pltpu.sync_copy(
            x_hbm.at[pl.ds(e_base + NMAIN * LOADW, TAILW), pl.ds(col, half)],
            xb0.at[pl.ds(0, TAILW)])
        pltpu.sync_copy(xb0.at[pl.ds(0, TAILW)], acc.at[idt_v], add=True)

        plsc.subcore_barrier()
        pltpu.sync_copy(
            acc.at[pl.ds(row0, rpt)],
            out_hbm.at[pl.ds(row0, rpt), pl.ds(c * half, half)])

        @pl.when(s == 15)
        def _():
            pltpu.sync_copy(
                acc.at[pl.ds(tail0 + rpt, tail_n - rpt)],
                out_hbm.at[pl.ds(tail0 + rpt, tail_n - rpt),
                           pl.ds(c * half, half)])

    return k(x, ids_main, ids_tail)


def kernel(h, m, rbf, id_j, W_rbf, W1_E, Wres_E, W_out_E,
           W1_F, Wres_F, W_out_F, scale_sum, scale_rbf):
    nAtoms = h.shape[0]

    ids = id_j.astype(jnp.int32).reshape(NCHUNKS, 16, EDGES_PER_TILE)
    wr16 = W_rbf.astype(jnp.bfloat16)
    rbf16 = rbf.astype(jnp.bfloat16)
    w1_f = (W1_F * (0.5 * scale_rbf)).astype(jnp.bfloat16)
    wres_f = (Wres_F * 0.5).astype(jnp.bfloat16)
    wout_f = W_out_F.astype(jnp.bfloat16)
    w1_e = (W1_E * (0.5 * scale_sum)).astype(jnp.bfloat16)
    wres_e = (Wres_E * 0.5).astype(jnp.bfloat16)
    wout_e = W_out_E.astype(jnp.bfloat16)

    xs_parts = []
    xf_parts = []
    for chunk in range(NCHUNKS):
        x_c, xf_c = _fused_stage(m, rbf16, wr16, w1_f, wres_f, wout_f,
                                 chunk, block=4000)
        ids_c = ids[chunk]
        ids_main = ids_c[:, :NMAIN * LOADW].reshape(16, NMAIN, LOADW)
        ids_tail = ids_c[:, NMAIN * LOADW:]
        xs_parts.append(_segsum(x_c, ids_main, ids_tail, nAtoms))
        xf_parts.append(xf_c)

    x_E = _energy_stage(xs_parts, w1_e, wres_e, wout_e, block=2000)
    x_F = jnp.concatenate(xf_parts, axis=0)
    return (x_E, x_F)
